# baseline (speedup 1.0000x reference)
import functools

import jax
import jax.numpy as jnp
from jax import lax
from jax.experimental import pallas as pl
from jax.experimental.pallas import tpu as pltpu
from jax.experimental.pallas import tpu_sc as plsc

N = 10000
D = 128
H = 64
C = 4

NPAD = 10240
ROWS_PER_TILE = NPAD // 16
NCORES = 2
NTILES = NCORES * 16


def _make_agg(e_pad, ntab):
    width = H
    chunk = 128
    ept = e_pad // NTILES
    cpt = ept // chunk
    nbuf = 4
    nslots = 2 * nbuf
    groups = cpt // nbuf
    assert cpt % (2 * nbuf) == 0
    mesh = plsc.VectorSubcoreMesh(core_axis_name="c", subcore_axis_name="s")

    @functools.partial(
        pl.kernel,
        out_type=jax.ShapeDtypeStruct((NCORES, ntab, NPAD, width),
                                      jnp.float32),
        mesh=mesh,
        scratch_types=[
            pltpu.VMEM((cpt, chunk), jnp.int32),
            pltpu.VMEM((cpt, chunk), jnp.int32),
            [pltpu.VMEM((chunk, width), jnp.float32)] * nslots,
            pltpu.VMEM_SHARED((NPAD, width), jnp.float32),
            [pltpu.SemaphoreType.DMA] * nslots,
            [pltpu.SemaphoreType.DMA] * nslots,
        ],
        compiler_params=pltpu.CompilerParams(use_tc_tiling_on_sc=False),
    )
    def agg(*args):
        ys = args[:ntab]
        src_hbm, dst_hbm, zero_hbm, out_hbm = args[ntab:ntab + 4]
        src_v, dst_v, rows_v, acc, gsem, ssem = args[ntab + 4:]
        cid = lax.axis_index("c")
        sid = lax.axis_index("s")
        tile = cid * 16 + sid

        pltpu.sync_copy(src_hbm.at[pl.ds(tile * cpt, cpt)], src_v)
        pltpu.sync_copy(dst_hbm.at[pl.ds(tile * cpt, cpt)], dst_v)

        for t in range(ntab):
            y_hbm = ys[t]

            @pl.when(sid == 0)
            def _():
                pltpu.sync_copy(zero_hbm, acc)

            plsc.subcore_barrier()

            def gather(j, slot):
                pltpu.async_copy(y_hbm.at[src_v.at[j]], rows_v[slot],
                                 gsem[slot])

            def gather_wait(j, slot):
                pltpu.make_async_copy(y_hbm.at[src_v.at[j]], rows_v[slot],
                                      gsem[slot]).wait()

            def scatter(j, slot):
                pltpu.async_copy(rows_v[slot], acc.at[dst_v.at[j]],
                                 ssem[slot], add=True)

            def scatter_wait(j, slot):
                pltpu.make_async_copy(rows_v[slot], acc.at[dst_v.at[j]],
                                      ssem[slot]).wait()

            for b in range(nbuf):
                gather(b, b)

            def phase(g, base):
                other = nbuf - base
                @pl.when(g >= 1)
                def _():
                    for b in range(nbuf):
                        scatter_wait(jnp.maximum(g - 1, 0) * nbuf + b,
                                     other + b)
                for b in range(nbuf):
                    gather(jnp.minimum((g + 1) * nbuf + b, cpt - 1),
                           other + b)
                for b in range(nbuf):
                    gather_wait(g * nbuf + b, base + b)
                    scatter(g * nbuf + b, base + b)

            def pair(p, carry):
                phase(2 * p, 0)
                phase(2 * p + 1, nbuf)
                return carry

            lax.fori_loop(0, groups // 2, pair, 0)
            for b in range(nbuf):
                gather_wait(cpt - 1, b)
                scatter_wait((groups - 1) * nbuf + b, nbuf + b)

            plsc.subcore_barrier()
            pltpu.sync_copy(
                acc.at[pl.ds(sid * ROWS_PER_TILE, ROWS_PER_TILE)],
                out_hbm.at[cid, t, pl.ds(sid * ROWS_PER_TILE, ROWS_PER_TILE)],
            )
            plsc.subcore_barrier()

    return agg


def _gin_mlp(pre, w1_ref, g_ref, beta_ref, w2_ref, b2_ref):
    h = jnp.dot(pre, w1_ref[...], preferred_element_type=jnp.float32)
    mu = jnp.sum(h, axis=0) / N
    dev = h - mu
    var = (jnp.sum(dev * dev, axis=0) - (NPAD - N) * mu * mu) / N
    hn = dev / jnp.sqrt(var + 1e-5) * g_ref[...] + beta_ref[...]
    a = jnp.maximum(hn, 0.0)
    return jnp.maximum(
        jnp.dot(a, w2_ref[...], preferred_element_type=jnp.float32)
        + b2_ref[...], 0.0)


def _mlp1_body(x_ref, agg_ref, w1_ref, g_ref, beta_ref, w2_ref, b2_ref,
               o_ref):
    agg = jnp.concatenate(
        [agg_ref[0, 0] + agg_ref[1, 0], agg_ref[0, 1] + agg_ref[1, 1]],
        axis=-1)
    pre = x_ref[...] + agg
    h1 = _gin_mlp(pre, w1_ref, g_ref, beta_ref, w2_ref, b2_ref)
    mask = lax.broadcasted_iota(jnp.int32, (NPAD, 1), 0) < N
    o_ref[...] = jnp.where(mask, h1, 0.0)


def _mlp2_head_body(h1_ref, agg_ref, w1_ref, g_ref, beta_ref, w2_ref, b2_ref,
                    fw1_ref, fb1_ref, fw2_ref, fb2_ref, o_ref):
    pre = h1_ref[...] + agg_ref[0] + agg_ref[1]
    h2 = _gin_mlp(pre, w1_ref, g_ref, beta_ref, w2_ref, b2_ref)
    t = jnp.maximum(
        jnp.dot(h2, fw1_ref[...], preferred_element_type=jnp.float32)
        + fb1_ref[...], 0.0)
    o_ref[...] = (jnp.dot(t, fw2_ref[...], preferred_element_type=jnp.float32)
                  + fb2_ref[...])


def kernel(x, edge_index, c1_W1, c1_b1, c1_g, c1_beta, c1_W2, c1_b2,
           c2_W1, c2_b1, c2_g, c2_beta, c2_W2, c2_b2,
           f_W1, f_b1, f_W2, f_b2):
    e = edge_index.shape[1]
    quantum = NTILES * 128 * 8
    e_pad = -(-e // quantum) * quantum
    nch = e_pad // 128

    x_pad = jnp.zeros((NPAD, D), jnp.float32).at[:N].set(x)
    x_lo = x_pad[:, :H]
    x_hi = x_pad[:, H:]
    pad_idx = jnp.full((e_pad - e,), N, jnp.int32)
    src2d = jnp.concatenate([edge_index[0], pad_idx]).reshape(nch, 128)
    dst2d = jnp.concatenate([edge_index[1], pad_idx]).reshape(nch, 128)
    zeros_h = jnp.zeros((NPAD, H), jnp.float32)

    agg1 = _make_agg(e_pad, 2)(x_lo, x_hi, src2d, dst2d, zeros_h)

    h1 = pl.pallas_call(
        _mlp1_body,
        out_shape=jax.ShapeDtypeStruct((NPAD, H), jnp.float32),
    )(x_pad, agg1, c1_W1, c1_g, c1_beta, c1_W2, c1_b2)

    agg2 = _make_agg(e_pad, 1)(h1, src2d, dst2d, zeros_h)[:, 0]

    out = pl.pallas_call(
        _mlp2_head_body,
        out_shape=jax.ShapeDtypeStruct((NPAD, C), jnp.float32),
    )(h1, agg2, c2_W1, c2_g, c2_beta, c2_W2, c2_b2, f_W1, f_b1, f_W2, f_b2)

    return out[:N]


# speedup vs baseline: 1.1378x; 1.1378x over previous
import functools

import jax
import jax.numpy as jnp
from jax import lax
from jax.experimental import pallas as pl
from jax.experimental.pallas import tpu as pltpu
from jax.experimental.pallas import tpu_sc as plsc

N = 10000
D = 128
H = 64
C = 4

NPAD = 10240
ROWS_PER_TILE = NPAD // 16
NCORES = 2
NTILES = NCORES * 16


def _make_agg(e_pad, width):
    chunk = 128 if width <= 64 else 64
    ept = e_pad // NTILES
    cpt = ept // chunk
    nbuf = 4 if width <= 64 else 2
    nslots = 2 * nbuf
    groups = cpt // nbuf
    assert cpt % (2 * nbuf) == 0
    mesh = plsc.VectorSubcoreMesh(core_axis_name="c", subcore_axis_name="s")

    @functools.partial(
        pl.kernel,
        out_type=jax.ShapeDtypeStruct((NCORES, NPAD, width), jnp.float32),
        mesh=mesh,
        scratch_types=[
            pltpu.VMEM((cpt, chunk), jnp.int32),
            pltpu.VMEM((cpt, chunk), jnp.int32),
            [pltpu.VMEM((chunk, width), jnp.float32)] * nslots,
            pltpu.VMEM_SHARED((NPAD, width), jnp.float32),
            [pltpu.SemaphoreType.DMA] * nslots,
        ],
        compiler_params=pltpu.CompilerParams(use_tc_tiling_on_sc=False),
    )
    def agg(y_hbm, src_hbm, dst_hbm, zero_hbm, out_hbm,
            src_v, dst_v, rows_v, acc, gsem):
        cid = lax.axis_index("c")
        sid = lax.axis_index("s")
        tile = cid * 16 + sid

        @pl.when(sid == 0)
        def _():
            pltpu.sync_copy(zero_hbm, acc)

        pltpu.sync_copy(src_hbm.at[pl.ds(tile * cpt, cpt)], src_v)
        pltpu.sync_copy(dst_hbm.at[pl.ds(tile * cpt, cpt)], dst_v)
        plsc.subcore_barrier()

        def gather(j, slot):
            pltpu.async_copy(y_hbm.at[src_v.at[j]], rows_v[slot], gsem[slot])

        def gather_wait(j, slot):
            pltpu.make_async_copy(y_hbm.at[src_v.at[j]], rows_v[slot],
                                  gsem[slot]).wait()

        def scatter(j, slot):
            pltpu.sync_copy(rows_v[slot], acc.at[dst_v.at[j]], add=True)

        for b in range(nbuf):
            gather(b, b)

        if width <= 64:
            def phase(g, base):
                other = nbuf - base
                for b in range(nbuf):
                    gather(jnp.minimum((g + 1) * nbuf + b, cpt - 1),
                           other + b)
                for b in range(nbuf):
                    gather_wait(g * nbuf + b, base + b)
                    scatter(g * nbuf + b, base + b)

            def pair(p, carry):
                phase(2 * p, 0)
                phase(2 * p + 1, nbuf)
                return carry

            lax.fori_loop(0, groups // 2, pair, 0)
            for b in range(nbuf):
                gather_wait(cpt - 1, b)
        else:
            def body(g, carry):
                for b in range(nbuf):
                    gather_wait(g * nbuf + b, b)
                    scatter(g * nbuf + b, b)
                    gather(jnp.minimum((g + 1) * nbuf + b, cpt - 1), b)
                return carry

            lax.fori_loop(0, groups, body, 0)
            for b in range(nbuf):
                gather_wait(cpt - 1, b)

        plsc.subcore_barrier()
        pltpu.sync_copy(
            acc.at[pl.ds(sid * ROWS_PER_TILE, ROWS_PER_TILE)],
            out_hbm.at[cid, pl.ds(sid * ROWS_PER_TILE, ROWS_PER_TILE)],
        )

    return agg


def _gin_mlp(pre, w1_ref, g_ref, beta_ref, w2_ref, b2_ref):
    h = jnp.dot(pre, w1_ref[...], preferred_element_type=jnp.float32)
    mu = jnp.sum(h, axis=0) / N
    dev = h - mu
    var = (jnp.sum(dev * dev, axis=0) - (NPAD - N) * mu * mu) / N
    hn = dev / jnp.sqrt(var + 1e-5) * g_ref[...] + beta_ref[...]
    a = jnp.maximum(hn, 0.0)
    return jnp.maximum(
        jnp.dot(a, w2_ref[...], preferred_element_type=jnp.float32)
        + b2_ref[...], 0.0)


def _mlp1_body(x_ref, agg_ref, w1_ref, g_ref, beta_ref, w2_ref, b2_ref,
               o_ref):
    pre = x_ref[...] + agg_ref[0] + agg_ref[1]
    h1 = _gin_mlp(pre, w1_ref, g_ref, beta_ref, w2_ref, b2_ref)
    mask = lax.broadcasted_iota(jnp.int32, (NPAD, 1), 0) < N
    o_ref[...] = jnp.where(mask, h1, 0.0)


def _mlp2_head_body(h1_ref, agg_ref, w1_ref, g_ref, beta_ref, w2_ref, b2_ref,
                    fw1_ref, fb1_ref, fw2_ref, fb2_ref, o_ref):
    pre = h1_ref[...] + agg_ref[0] + agg_ref[1]
    h2 = _gin_mlp(pre, w1_ref, g_ref, beta_ref, w2_ref, b2_ref)
    t = jnp.maximum(
        jnp.dot(h2, fw1_ref[...], preferred_element_type=jnp.float32)
        + fb1_ref[...], 0.0)
    o_ref[...] = (jnp.dot(t, fw2_ref[...], preferred_element_type=jnp.float32)
                  + fb2_ref[...])


def kernel(x, edge_index, c1_W1, c1_b1, c1_g, c1_beta, c1_W2, c1_b2,
           c2_W1, c2_b1, c2_g, c2_beta, c2_W2, c2_b2,
           f_W1, f_b1, f_W2, f_b2):
    e = edge_index.shape[1]
    quantum = NTILES * 128 * 8
    e_pad = -(-e // quantum) * quantum
    nch = e_pad // 128

    x_pad = jnp.zeros((NPAD, D), jnp.float32).at[:N].set(x)
    pad_idx = jnp.full((e_pad - e,), N, jnp.int32)
    src_flat = jnp.concatenate([edge_index[0], pad_idx])
    dst_flat = jnp.concatenate([edge_index[1], pad_idx])
    zeros_d = jnp.zeros((NPAD, D), jnp.float32)
    zeros_h = jnp.zeros((NPAD, H), jnp.float32)

    agg1 = _make_agg(e_pad, D)(x_pad, src_flat.reshape(-1, 64),
                               dst_flat.reshape(-1, 64), zeros_d)

    h1 = pl.pallas_call(
        _mlp1_body,
        out_shape=jax.ShapeDtypeStruct((NPAD, H), jnp.float32),
    )(x_pad, agg1, c1_W1, c1_g, c1_beta, c1_W2, c1_b2)

    agg2 = _make_agg(e_pad, H)(h1, src_flat.reshape(-1, 128),
                               dst_flat.reshape(-1, 128), zeros_h)

    out = pl.pallas_call(
        _mlp2_head_body,
        out_shape=jax.ShapeDtypeStruct((NPAD, C), jnp.float32),
    )(h1, agg2, c2_W1, c2_g, c2_beta, c2_W2, c2_b2, f_W1, f_b1, f_W2, f_b2)

    return out[:N]
